# Initial kernel scaffold; baseline (speedup 1.0000x reference)
import jax
import jax.numpy as jnp
from jax.experimental import pallas as pl


def kernel(x):
    raise NotImplementedError("write your pallas kernel here")


# speedup vs baseline: 2.4366x; 2.4366x over previous
import functools

import jax
import jax.numpy as jnp
import numpy as np
from jax.experimental import pallas as pl
from jax.experimental.pallas import tpu as pltpu


@functools.lru_cache(maxsize=None)
def _half_sel_matrix(W):
    W_out = W // 2
    sel = np.zeros((16 * W, 8 * W_out), dtype=np.float32)
    hl = np.repeat(np.arange(0, 16, 2), W_out)
    w = np.tile(np.arange(0, W, 2), 8)
    rows = hl * W + w
    cols = (hl // 2) * W_out + (w // 2)
    sel[rows, cols] = 1.0
    return sel


def _make_body(Nb, C, pad):
    C_out2 = 2 * (C + 2 * pad)

    def body(x_ref, s_ref, o_ref):
        y = jnp.dot(
            x_ref[...].reshape(Nb * 2 * C, 512),
            s_ref[...],
            preferred_element_type=jnp.float32,
        )
        o_ref[:, : 2 * pad, :] = jnp.zeros((Nb, 2 * pad, 128), o_ref.dtype)
        o_ref[:, 2 * pad : 2 * (pad + C), :] = y.reshape(Nb, 2 * C, 128)
        o_ref[:, 2 * (pad + C) :, :] = jnp.zeros((Nb, 2 * pad, 128), o_ref.dtype)

    return body


def _lambda_layer(x, planes):
    N, C, H, W = x.shape
    pad = planes // 4
    H_out, W_out = H // 2, W // 2
    C_out = C + 2 * pad

    halves = H // 16
    x3 = x.reshape(N, C * halves, 16 * W)
    sel = jnp.asarray(_half_sel_matrix(W))

    Nb = 64
    while N % Nb:
        Nb //= 2
    C_out2 = halves * C_out

    itemsize = 4
    cost = pl.CostEstimate(
        flops=2 * (N * C * halves) * (16 * W) * (8 * W_out),
        transcendentals=0,
        bytes_accessed=(x.size + sel.size + N * C_out2 * 8 * W_out) * itemsize,
    )

    out = pl.pallas_call(
        _make_body(Nb, C, pad),
        out_shape=jax.ShapeDtypeStruct((N, C_out2, 8 * W_out), x.dtype),
        grid=(N // Nb,),
        in_specs=[
            pl.BlockSpec((Nb, C * halves, 16 * W), lambda n: (n, 0, 0)),
            pl.BlockSpec((16 * W, 8 * W_out), lambda n: (0, 0)),
        ],
        out_specs=pl.BlockSpec((Nb, C_out2, 8 * W_out), lambda n: (n, 0, 0)),
        compiler_params=pltpu.CompilerParams(
            dimension_semantics=("parallel",),
            vmem_limit_bytes=48 << 20,
        ),
        cost_estimate=cost,
    )(x3, sel)

    return out.reshape(N, C_out, H_out, W_out)


def kernel(x):
    return _lambda_layer(x, planes=32)
